# baseline (speedup 1.0000x reference)
import math

import jax
import jax.numpy as jnp
import numpy as np
from jax import lax
from jax.experimental import pallas as pl
from jax.experimental.pallas import tpu as pltpu
from jax.experimental.pallas import tpu_sc as plsc

_B, _C, _H, _W = 8, 96, 56, 56
_K = 9
_HR, _WR = 7, 7
_N = _H * _W
_NR = _HR * _WR
_GROUPS = 4
_BIG = 3.0e38

_NC, _NS, _L = 2, 16, 16
_NW = _NC * _NS


def _pos_embed_np(c, h, w):
    d = c // 2
    pe = np.zeros((c, h, w), dtype=np.float32)
    div = np.exp(np.arange(0.0, d, 2) * -(math.log(10000.0) / d))
    pos_w = np.arange(0.0, w)[:, None]
    pos_h = np.arange(0.0, h)[:, None]
    pe[0:d:2, :, :] = np.sin(pos_w * div).T[:, None, :]
    pe[1:d:2, :, :] = np.cos(pos_w * div).T[:, None, :]
    pe[d::2, :, :] = np.sin(pos_h * div).T[:, :, None]
    pe[d + 1::2, :, :] = np.cos(pos_h * div).T[:, :, None]
    return pe


def _constants():
    pos = _pos_embed_np(_C, _H, _W)
    pos_red = pos.reshape(_C, _HR, _H // _HR, _WR, _W // _WR).mean(axis=(2, 4))
    rel = 2.0 * (pos.reshape(_C, -1).T @ pos_red.reshape(_C, -1)) / _C
    relT = np.ascontiguousarray(rel.T).astype(np.float32)
    hh = np.arange(_H)[:, None]
    ww = np.arange(_W)[None, :]
    blk = (hh // (_H // _HR)) * _WR + (ww // (_W // _WR))
    pool = (blk.reshape(_N, 1) == np.arange(_NR)[None, :]).astype(np.float32) / 64.0
    return relT, pool


_RELT_NP, _POOL_NP = _constants()


def _make_tc1_body(wpi, npw, npw_pad):
    def _tc1_body(x_ref, w1_ref, b1_ref, relT_ref, pool_ref,
                  h_ref, yT_ref, idx_ref):
        x = x_ref[0]
        h = jnp.dot(w1_ref[...], x,
                    preferred_element_type=jnp.float32) + b1_ref[...]
        h_ref[0] = h

        y = jnp.dot(h, pool_ref[...], preferred_element_type=jnp.float32)
        yT_ref[0] = y

        nx = jnp.sqrt(jnp.sum(h * h, axis=0, keepdims=True))
        ny = jnp.sqrt(jnp.sum(y * y, axis=0, keepdims=True))
        ipT = jax.lax.dot_general(y, h, (((0,), (0,)), ((), ())),
                                  preferred_element_type=jnp.float32)
        inv_x = 1.0 / (nx + 1e-12)
        inv_y = 1.0 / (ny + 1e-12)
        innerT = ipT * inv_x * inv_y.reshape(_NR, 1)
        sx = (nx * inv_x) ** 2
        sy = (ny * inv_y) ** 2
        distT = 2.0 * innerT - sx - sy.reshape(_NR, 1) + relT_ref[...]

        iota0 = jax.lax.broadcasted_iota(jnp.int32, (_NR, _N), 0)
        d = distT
        rows = []
        for _ in range(_K):
            cur = jnp.max(d, axis=0, keepdims=True)
            first = jnp.min(jnp.where(d >= cur, iota0, _NR), axis=0,
                            keepdims=True)
            rows.append(first)
            d = jnp.where(iota0 == first, -_BIG, d)
        idxmat = jnp.concatenate(rows, axis=0)
        pad = jnp.zeros((_K, npw_pad - npw), jnp.int32)
        for q in range(wpi):
            idx_ref[0, q] = jnp.concatenate(
                [idxmat[:, q * npw:(q + 1) * npw], pad], axis=1)
    return _tc1_body


def _make_sc_body(wpi, npw_pad):
    ngrp = npw_pad // _L

    def _sc_body(yT_hbm, idx_hbm, out_hbm, yT_v, idx_v, acc_v):
        wid = lax.axis_index("s") * _NC + lax.axis_index("c")
        b = wid // wpi
        q = wid % wpi

        pltpu.sync_copy(yT_hbm.at[b], yT_v)
        pltpu.sync_copy(idx_hbm.at[b, q], idx_v)

        cu = 8

        def grp_body(g, carry):
            base = g * _L
            ivs = [idx_v[k, pl.ds(base, _L)] for k in range(_K)]

            def c_body(cc, carry2):
                c0 = cc * cu
                for u in range(cu):
                    csplat = jnp.full((_L,), 0, jnp.int32) + (c0 + u)
                    o = plsc.load_gather(yT_v, [csplat, ivs[0]])
                    for k in range(1, _K):
                        o = jnp.maximum(
                            o, plsc.load_gather(yT_v, [csplat, ivs[k]]))
                    acc_v[c0 + u, pl.ds(base, _L)] = o
                return carry2

            return lax.fori_loop(0, _C // cu, c_body, carry)

        lax.fori_loop(0, ngrp, grp_body, 0)
        pltpu.sync_copy(acc_v, out_hbm.at[b, q])
    return _sc_body


def _make_tc2_body(wpi, npw):
    def _tc2_body(h_ref, acc_ref, x_ref, mrw_ref, mrb_ref, mrg_ref,
                  mrbeta_ref, w2_ref, b2_ref, out_ref):
        h = h_ref[0]
        acc = jnp.concatenate(
            [acc_ref[0, q, :, :npw] for q in range(wpi)], axis=1)
        cat = jnp.concatenate([h, acc - h], axis=0)
        g = jnp.dot(mrw_ref[...], cat,
                    preferred_element_type=jnp.float32) + mrb_ref[...]

        rows = (2 * _C) // _GROUPS
        parts = []
        for gi in range(_GROUPS):
            sub = g[gi * rows:(gi + 1) * rows, :]
            m = jnp.mean(sub)
            dsub = sub - m
            v = jnp.mean(dsub * dsub)
            parts.append(dsub * jax.lax.rsqrt(v + 1e-5))
        gn = jnp.concatenate(parts, axis=0) * mrg_ref[...] + mrbeta_ref[...]
        act = jax.nn.gelu(gn)

        out_ref[0] = (jnp.dot(w2_ref[...], act,
                              preferred_element_type=jnp.float32)
                      + b2_ref[...] + x_ref[0])
    return _tc2_body


def _half_pipeline(xh, w1, b1, relT, pool, mr_w, mrb, mrg, mrbeta, w2, b2):
    bh = xh.shape[0]
    wpi = _NW // bh
    npw = _N // wpi
    npw_pad = -(-npw // _L) * _L

    full = lambda shape: pl.BlockSpec(shape, lambda b: (0,) * len(shape))
    batch = lambda shape: pl.BlockSpec((1,) + shape,
                                       lambda b: (b,) + (0,) * len(shape))

    h, yT, idx = pl.pallas_call(
        _make_tc1_body(wpi, npw, npw_pad),
        grid=(bh,),
        in_specs=[
            batch((_C, _N)),
            full((_C, _C)), full((_C, 1)),
            full((_NR, _N)), full((_N, _NR)),
        ],
        out_specs=[batch((_C, _N)), batch((_C, _NR)),
                   batch((wpi, _K, npw_pad))],
        out_shape=[
            jax.ShapeDtypeStruct((bh, _C, _N), jnp.float32),
            jax.ShapeDtypeStruct((bh, _C, _NR), jnp.float32),
            jax.ShapeDtypeStruct((bh, wpi, _K, npw_pad), jnp.int32),
        ],
    )(xh, w1, b1, relT, pool)

    mesh = plsc.VectorSubcoreMesh(core_axis_name="c", subcore_axis_name="s")
    acc = pl.kernel(
        _make_sc_body(wpi, npw_pad), mesh=mesh,
        compiler_params=pltpu.CompilerParams(needs_layout_passes=False),
        out_type=jax.ShapeDtypeStruct((bh, wpi, _C, npw_pad), jnp.float32),
        scratch_types=[
            pltpu.VMEM((_C, _NR), jnp.float32),
            pltpu.VMEM((_K, npw_pad), jnp.int32),
            pltpu.VMEM((_C, npw_pad), jnp.float32),
        ],
    )(yT, idx)

    return pl.pallas_call(
        _make_tc2_body(wpi, npw),
        grid=(bh,),
        in_specs=[
            batch((_C, _N)),
            batch((wpi, _C, npw_pad)),
            batch((_C, _N)),
            full((2 * _C, 2 * _C)), full((2 * _C, 1)),
            full((2 * _C, 1)), full((2 * _C, 1)),
            full((_C, 2 * _C)), full((_C, 1)),
        ],
        out_specs=batch((_C, _N)),
        out_shape=jax.ShapeDtypeStruct((bh, _C, _N), jnp.float32),
    )(h, acc, xh, mr_w, mrb, mrg, mrbeta, w2, b2)


def kernel(x, fc1_w, fc1_b, fc1_g, fc1_beta, mr_w, mr_b, mr_g, mr_beta,
           fc2_w, fc2_b, fc2_g, fc2_beta):
    x3 = x.reshape(_B, _C, _N)
    w1 = fc1_g[:, None] * fc1_w
    b1 = (fc1_g * fc1_b + fc1_beta)[:, None]
    w2 = fc2_g[:, None] * fc2_w
    b2 = (fc2_g * fc2_b + fc2_beta)[:, None]
    relT = jnp.asarray(_RELT_NP)
    pool = jnp.asarray(_POOL_NP)

    args = (w1, b1, relT, pool, mr_w, mr_b[:, None], mr_g[:, None],
            mr_beta[:, None], w2, b2)
    hb = _B // 2
    out_a = _half_pipeline(x3[:hb], *args)
    out_b = _half_pipeline(x3[hb:], *args)
    out = jnp.concatenate([out_a, out_b], axis=0)
    return out.reshape(_B, _C, _H, _W)


# speedup vs baseline: 1.2419x; 1.2419x over previous
import math

import jax
import jax.numpy as jnp
import numpy as np
from jax import lax
from jax.experimental import pallas as pl
from jax.experimental.pallas import tpu as pltpu
from jax.experimental.pallas import tpu_sc as plsc

_B, _C, _H, _W = 8, 96, 56, 56
_K = 9
_HR, _WR = 7, 7
_N = _H * _W
_NR = _HR * _WR
_GROUPS = 4
_BIG = 3.0e38

_NC, _NS, _L = 2, 16, 16
_NW = _NC * _NS

_HB = _B // 2
_WPI = _NW // _HB
_NPW = _N // _WPI
_NPWP = -(-_NPW // _L) * _L


def _pos_embed_np(c, h, w):
    d = c // 2
    pe = np.zeros((c, h, w), dtype=np.float32)
    div = np.exp(np.arange(0.0, d, 2) * -(math.log(10000.0) / d))
    pos_w = np.arange(0.0, w)[:, None]
    pos_h = np.arange(0.0, h)[:, None]
    pe[0:d:2, :, :] = np.sin(pos_w * div).T[:, None, :]
    pe[1:d:2, :, :] = np.cos(pos_w * div).T[:, None, :]
    pe[d::2, :, :] = np.sin(pos_h * div).T[:, :, None]
    pe[d + 1::2, :, :] = np.cos(pos_h * div).T[:, :, None]
    return pe


def _constants():
    pos = _pos_embed_np(_C, _H, _W)
    pos_red = pos.reshape(_C, _HR, _H // _HR, _WR, _W // _WR).mean(axis=(2, 4))
    rel = 2.0 * (pos.reshape(_C, -1).T @ pos_red.reshape(_C, -1)) / _C
    relT = np.ascontiguousarray(rel.T).astype(np.float32)
    hh = np.arange(_H)[:, None]
    ww = np.arange(_W)[None, :]
    blk = (hh // (_H // _HR)) * _WR + (ww // (_W // _WR))
    pool = (blk.reshape(_N, 1) == np.arange(_NR)[None, :]).astype(np.float32) / 64.0
    return relT, pool


_RELT_NP, _POOL_NP = _constants()


def _tc1_body(x_ref, w1_ref, b1_ref, relT_ref, pool_ref,
              h_ref, yT_ref, idx_ref):
    x = x_ref[0]
    h = jnp.dot(w1_ref[...], x,
                preferred_element_type=jnp.float32) + b1_ref[...]
    h_ref[0] = h

    y = jnp.dot(h, pool_ref[...], preferred_element_type=jnp.float32)
    yT_ref[0] = y

    nx = jnp.sqrt(jnp.sum(h * h, axis=0, keepdims=True))
    ny = jnp.sqrt(jnp.sum(y * y, axis=0, keepdims=True))
    ipT = jax.lax.dot_general(y, h, (((0,), (0,)), ((), ())),
                              preferred_element_type=jnp.float32)
    inv_x = 1.0 / (nx + 1e-12)
    inv_y = 1.0 / (ny + 1e-12)
    innerT = ipT * inv_x * inv_y.reshape(_NR, 1)
    sx = (nx * inv_x) ** 2
    sy = (ny * inv_y) ** 2
    distT = 2.0 * innerT - sx - sy.reshape(_NR, 1) + relT_ref[...]

    iota0 = jax.lax.broadcasted_iota(jnp.int32, (_NR, _N), 0)
    d = distT
    rows = []
    for _ in range(_K):
        cur = jnp.max(d, axis=0, keepdims=True)
        first = jnp.min(jnp.where(d >= cur, iota0, _NR), axis=0,
                        keepdims=True)
        rows.append(first)
        d = jnp.where(iota0 == first, -_BIG, d)
    idxmat = jnp.concatenate(rows, axis=0)
    pad = jnp.zeros((_K, _NPWP - _NPW), jnp.int32)
    for q in range(_WPI):
        idx_ref[0, q] = jnp.concatenate(
            [idxmat[:, q * _NPW:(q + 1) * _NPW], pad], axis=1)


def _sc_body(yT_hbm, idx_hbm, out_hbm, yT_v, idx_v, acc_v):
    wid = lax.axis_index("s") * _NC + lax.axis_index("c")
    b = wid // _WPI
    q = wid % _WPI

    pltpu.sync_copy(yT_hbm.at[b], yT_v)
    pltpu.sync_copy(idx_hbm.at[b, q], idx_v)

    cu = 8

    def grp_body(g, carry):
        base = g * _L
        ivs = [idx_v[k, pl.ds(base, _L)] for k in range(_K)]

        def c_body(cc, carry2):
            c0 = cc * cu
            for u in range(cu):
                csplat = jnp.full((_L,), 0, jnp.int32) + (c0 + u)
                o = plsc.load_gather(yT_v, [csplat, ivs[0]])
                for k in range(1, _K):
                    o = jnp.maximum(
                        o, plsc.load_gather(yT_v, [csplat, ivs[k]]))
                acc_v[c0 + u, pl.ds(base, _L)] = o
            return carry2

        return lax.fori_loop(0, _C // cu, c_body, carry)

    lax.fori_loop(0, _NPWP // _L, grp_body, 0)
    pltpu.sync_copy(acc_v, out_hbm.at[b, q])


def _tc2_compute(h, acc_ref, x, mrw_ref, mrb_ref, mrg_ref, mrbeta_ref,
                 w2_ref, b2_ref):
    acc = jnp.concatenate(
        [acc_ref[0, q, :, :_NPW] for q in range(_WPI)], axis=1)
    cat = jnp.concatenate([h, acc - h], axis=0)
    g = jnp.dot(mrw_ref[...], cat,
                preferred_element_type=jnp.float32) + mrb_ref[...]

    rows = (2 * _C) // _GROUPS
    parts = []
    for gi in range(_GROUPS):
        sub = g[gi * rows:(gi + 1) * rows, :]
        m = jnp.mean(sub)
        dsub = sub - m
        v = jnp.mean(dsub * dsub)
        parts.append(dsub * jax.lax.rsqrt(v + 1e-5))
    gn = jnp.concatenate(parts, axis=0) * mrg_ref[...] + mrbeta_ref[...]
    act = jax.nn.gelu(gn)

    return (jnp.dot(w2_ref[...], act, preferred_element_type=jnp.float32)
            + b2_ref[...] + x)


def _tc2a_body(h_ref, acc_ref, x_ref, mrw_ref, mrb_ref, mrg_ref,
               mrbeta_ref, w2_ref, b2_ref, out_ref):
    out_ref[0] = _tc2_compute(h_ref[0], acc_ref, x_ref[0], mrw_ref, mrb_ref,
                              mrg_ref, mrbeta_ref, w2_ref, b2_ref)


def _tc2b_body(outa_ref, h_ref, acc_ref, x_ref, mrw_ref, mrb_ref, mrg_ref,
               mrbeta_ref, w2_ref, b2_ref, out_ref):
    b = pl.program_id(0)

    @pl.when(b < _HB)
    def _copy():
        out_ref[0] = outa_ref[0]

    @pl.when(b >= _HB)
    def _compute():
        out_ref[0] = _tc2_compute(h_ref[0], acc_ref, x_ref[0], mrw_ref,
                                  mrb_ref, mrg_ref, mrbeta_ref, w2_ref,
                                  b2_ref)


def kernel(x, fc1_w, fc1_b, fc1_g, fc1_beta, mr_w, mr_b, mr_g, mr_beta,
           fc2_w, fc2_b, fc2_g, fc2_beta):
    x3 = x.reshape(_B, _C, _N)
    w1 = fc1_g[:, None] * fc1_w
    b1 = (fc1_g * fc1_b + fc1_beta)[:, None]
    w2 = fc2_g[:, None] * fc2_w
    b2 = (fc2_g * fc2_b + fc2_beta)[:, None]
    relT = jnp.asarray(_RELT_NP)
    pool = jnp.asarray(_POOL_NP)

    full = lambda shape: pl.BlockSpec(shape, lambda b: (0,) * len(shape))

    def tc1(off):
        return pl.pallas_call(
            _tc1_body,
            grid=(_HB,),
            in_specs=[
                pl.BlockSpec((1, _C, _N), lambda b: (b + off, 0, 0)),
                full((_C, _C)), full((_C, 1)),
                full((_NR, _N)), full((_N, _NR)),
            ],
            out_specs=[
                pl.BlockSpec((1, _C, _N), lambda b: (b, 0, 0)),
                pl.BlockSpec((1, _C, _NR), lambda b: (b, 0, 0)),
                pl.BlockSpec((1, _WPI, _K, _NPWP), lambda b: (b, 0, 0, 0)),
            ],
            out_shape=[
                jax.ShapeDtypeStruct((_HB, _C, _N), jnp.float32),
                jax.ShapeDtypeStruct((_HB, _C, _NR), jnp.float32),
                jax.ShapeDtypeStruct((_HB, _WPI, _K, _NPWP), jnp.int32),
            ],
        )(x3, w1, b1, relT, pool)

    mesh = plsc.VectorSubcoreMesh(core_axis_name="c", subcore_axis_name="s")

    def sc(yT, idx):
        return pl.kernel(
            _sc_body, mesh=mesh,
            compiler_params=pltpu.CompilerParams(needs_layout_passes=False),
            out_type=jax.ShapeDtypeStruct((_HB, _WPI, _C, _NPWP),
                                          jnp.float32),
            scratch_types=[
                pltpu.VMEM((_C, _NR), jnp.float32),
                pltpu.VMEM((_K, _NPWP), jnp.int32),
                pltpu.VMEM((_C, _NPWP), jnp.float32),
            ],
        )(yT, idx)

    h_a, yT_a, idx_a = tc1(0)
    h_b, yT_b, idx_b = tc1(_HB)
    acc_a = sc(yT_a, idx_a)
    acc_b = sc(yT_b, idx_b)

    w_specs = [full((2 * _C, 2 * _C)), full((2 * _C, 1)),
               full((2 * _C, 1)), full((2 * _C, 1)),
               full((_C, 2 * _C)), full((_C, 1))]
    w_args = (mr_w, mr_b[:, None], mr_g[:, None], mr_beta[:, None], w2, b2)

    out_a = pl.pallas_call(
        _tc2a_body,
        grid=(_HB,),
        in_specs=[
            pl.BlockSpec((1, _C, _N), lambda b: (b, 0, 0)),
            pl.BlockSpec((1, _WPI, _C, _NPWP), lambda b: (b, 0, 0, 0)),
            pl.BlockSpec((1, _C, _N), lambda b: (b, 0, 0)),
        ] + w_specs,
        out_specs=pl.BlockSpec((1, _C, _N), lambda b: (b, 0, 0)),
        out_shape=jax.ShapeDtypeStruct((_HB, _C, _N), jnp.float32),
    )(h_a, acc_a, x3, *w_args)

    out = pl.pallas_call(
        _tc2b_body,
        grid=(_B,),
        in_specs=[
            pl.BlockSpec((1, _C, _N),
                         lambda b: (jnp.minimum(b, _HB - 1), 0, 0)),
            pl.BlockSpec((1, _C, _N),
                         lambda b: (jnp.maximum(b - _HB, 0), 0, 0)),
            pl.BlockSpec((1, _WPI, _C, _NPWP),
                         lambda b: (jnp.maximum(b - _HB, 0), 0, 0, 0)),
            pl.BlockSpec((1, _C, _N), lambda b: (b, 0, 0)),
        ] + w_specs,
        out_specs=pl.BlockSpec((1, _C, _N), lambda b: (b, 0, 0)),
        out_shape=jax.ShapeDtypeStruct((_B, _C, _N), jnp.float32),
    )(out_a, h_b, acc_b, x3, *w_args)

    return out.reshape(_B, _C, _H, _W)


# speedup vs baseline: 1.4893x; 1.1992x over previous
import math

import jax
import jax.numpy as jnp
import numpy as np
from jax import lax
from jax.experimental import pallas as pl
from jax.experimental.pallas import tpu as pltpu
from jax.experimental.pallas import tpu_sc as plsc

_B, _C, _H, _W = 8, 96, 56, 56
_K = 9
_HR, _WR = 7, 7
_N = _H * _W
_NR = _HR * _WR
_GROUPS = 4
_BIG = 3.0e38

_NC, _NS, _L = 2, 16, 16
_NW = _NC * _NS

_HB = _B // 2
_WPI = _NW // _HB
_NPW = _N // _WPI
_NPWP = -(-_NPW // _L) * _L


def _pos_embed_np(c, h, w):
    d = c // 2
    pe = np.zeros((c, h, w), dtype=np.float32)
    div = np.exp(np.arange(0.0, d, 2) * -(math.log(10000.0) / d))
    pos_w = np.arange(0.0, w)[:, None]
    pos_h = np.arange(0.0, h)[:, None]
    pe[0:d:2, :, :] = np.sin(pos_w * div).T[:, None, :]
    pe[1:d:2, :, :] = np.cos(pos_w * div).T[:, None, :]
    pe[d::2, :, :] = np.sin(pos_h * div).T[:, :, None]
    pe[d + 1::2, :, :] = np.cos(pos_h * div).T[:, :, None]
    return pe


def _constants():
    pos = _pos_embed_np(_C, _H, _W)
    pos_red = pos.reshape(_C, _HR, _H // _HR, _WR, _W // _WR).mean(axis=(2, 4))
    rel = 2.0 * (pos.reshape(_C, -1).T @ pos_red.reshape(_C, -1)) / _C
    relT = np.ascontiguousarray(rel.T).astype(np.float32)
    hh = np.arange(_H)[:, None]
    ww = np.arange(_W)[None, :]
    blk = (hh // (_H // _HR)) * _WR + (ww // (_W // _WR))
    pool = (blk.reshape(_N, 1) == np.arange(_NR)[None, :]).astype(np.float32) / 64.0
    return relT, pool


_RELT_NP, _POOL_NP = _constants()


def _tc1_body(x_ref, w1_ref, b1_ref, relT_ref, pool_ref,
              h_ref, yT_ref, idx_ref):
    x = x_ref[0]
    h = jnp.dot(w1_ref[...], x,
                preferred_element_type=jnp.float32) + b1_ref[...]
    h_ref[0] = h

    y = jnp.dot(h, pool_ref[...], preferred_element_type=jnp.float32)
    yr = y.astype(jnp.bfloat16).reshape(_C // 2, 2, _NR)
    ue = jax.lax.bitcast_convert_type(yr[:, 0, :], jnp.uint16).astype(jnp.uint32)
    uo = jax.lax.bitcast_convert_type(yr[:, 1, :], jnp.uint16).astype(jnp.uint32)
    yT_ref[0] = jax.lax.bitcast_convert_type((uo << 16) | ue, jnp.int32)

    nx = jnp.sqrt(jnp.sum(h * h, axis=0, keepdims=True))
    ny = jnp.sqrt(jnp.sum(y * y, axis=0, keepdims=True))
    ipT = jax.lax.dot_general(y, h, (((0,), (0,)), ((), ())),
                              preferred_element_type=jnp.float32)
    inv_x = 1.0 / (nx + 1e-12)
    inv_y = 1.0 / (ny + 1e-12)
    innerT = ipT * inv_x * inv_y.reshape(_NR, 1)
    sx = (nx * inv_x) ** 2
    sy = (ny * inv_y) ** 2
    distT = 2.0 * innerT - sx - sy.reshape(_NR, 1) + relT_ref[...]

    iota0 = jax.lax.broadcasted_iota(jnp.int32, (_NR, _N), 0)
    d = distT
    rows = []
    for _ in range(_K):
        cur = jnp.max(d, axis=0, keepdims=True)
        first = jnp.min(jnp.where(d >= cur, iota0, _NR), axis=0,
                        keepdims=True)
        rows.append(first)
        d = jnp.where(iota0 == first, -_BIG, d)
    idxmat = jnp.concatenate(rows, axis=0)
    pad = jnp.zeros((_K, _NPWP - _NPW), jnp.int32)
    for q in range(_WPI):
        idx_ref[0, q] = jnp.concatenate(
            [idxmat[:, q * _NPW:(q + 1) * _NPW], pad], axis=1)


def _sc_body(yT_hbm, idx_hbm, out_hbm, yT_v, idx_v, acc_v):
    wid = lax.axis_index("s") * _NC + lax.axis_index("c")
    b = wid // _WPI
    q = wid % _WPI

    pltpu.sync_copy(yT_hbm.at[b], yT_v)
    pltpu.sync_copy(idx_hbm.at[b, q], idx_v)

    cu = 8

    def grp_body(g, carry):
        base = g * _L
        ivs = [idx_v[k, pl.ds(base, _L)] for k in range(_K)]

        def c_body(cc, carry2):
            c0 = cc * cu
            for u in range(cu):
                cp = c0 + u
                csplat = jnp.full((_L,), 0, jnp.int32) + cp
                o = plsc.bitcast(plsc.load_gather(yT_v, [csplat, ivs[0]]),
                                 jnp.bfloat16)
                for k in range(1, _K):
                    o = jnp.maximum(o, plsc.bitcast(
                        plsc.load_gather(yT_v, [csplat, ivs[k]]),
                        jnp.bfloat16))
                ev, od = plsc.unpack(o, format=plsc.PackFormat.INTERLEAVED)
                acc_v[2 * cp, pl.ds(base, _L)] = ev
                acc_v[2 * cp + 1, pl.ds(base, _L)] = od
            return carry2

        return lax.fori_loop(0, (_C // 2) // cu, c_body, carry)

    lax.fori_loop(0, _NPWP // _L, grp_body, 0)
    pltpu.sync_copy(acc_v, out_hbm.at[b, q])


def _tc2_compute(h, acc_ref, x, mrw_ref, mrb_ref, mrg_ref, mrbeta_ref,
                 w2_ref, b2_ref):
    acc = jnp.concatenate(
        [acc_ref[0, q, :, :_NPW] for q in range(_WPI)], axis=1)
    cat = jnp.concatenate([h, acc - h], axis=0)
    g = jnp.dot(mrw_ref[...], cat,
                preferred_element_type=jnp.float32) + mrb_ref[...]

    rows = (2 * _C) // _GROUPS
    parts = []
    for gi in range(_GROUPS):
        sub = g[gi * rows:(gi + 1) * rows, :]
        m = jnp.mean(sub)
        dsub = sub - m
        v = jnp.mean(dsub * dsub)
        parts.append(dsub * jax.lax.rsqrt(v + 1e-5))
    gn = jnp.concatenate(parts, axis=0) * mrg_ref[...] + mrbeta_ref[...]
    act = jax.nn.gelu(gn)

    return (jnp.dot(w2_ref[...], act, preferred_element_type=jnp.float32)
            + b2_ref[...] + x)


def _tc2a_body(h_ref, acc_ref, x_ref, mrw_ref, mrb_ref, mrg_ref,
               mrbeta_ref, w2_ref, b2_ref, out_ref):
    out_ref[0] = _tc2_compute(h_ref[0], acc_ref, x_ref[0], mrw_ref, mrb_ref,
                              mrg_ref, mrbeta_ref, w2_ref, b2_ref)


def _tc2b_body(outa_ref, h_ref, acc_ref, x_ref, mrw_ref, mrb_ref, mrg_ref,
               mrbeta_ref, w2_ref, b2_ref, out_ref):
    b = pl.program_id(0)

    @pl.when(b < _HB)
    def _copy():
        out_ref[0] = outa_ref[0]

    @pl.when(b >= _HB)
    def _compute():
        out_ref[0] = _tc2_compute(h_ref[0], acc_ref, x_ref[0], mrw_ref,
                                  mrb_ref, mrg_ref, mrbeta_ref, w2_ref,
                                  b2_ref)


def kernel(x, fc1_w, fc1_b, fc1_g, fc1_beta, mr_w, mr_b, mr_g, mr_beta,
           fc2_w, fc2_b, fc2_g, fc2_beta):
    x3 = x.reshape(_B, _C, _N)
    w1 = fc1_g[:, None] * fc1_w
    b1 = (fc1_g * fc1_b + fc1_beta)[:, None]
    w2 = fc2_g[:, None] * fc2_w
    b2 = (fc2_g * fc2_b + fc2_beta)[:, None]
    relT = jnp.asarray(_RELT_NP)
    pool = jnp.asarray(_POOL_NP)

    full = lambda shape: pl.BlockSpec(shape, lambda b: (0,) * len(shape))

    def tc1(off):
        return pl.pallas_call(
            _tc1_body,
            grid=(_HB,),
            in_specs=[
                pl.BlockSpec((1, _C, _N), lambda b: (b + off, 0, 0)),
                full((_C, _C)), full((_C, 1)),
                full((_NR, _N)), full((_N, _NR)),
            ],
            out_specs=[
                pl.BlockSpec((1, _C, _N), lambda b: (b, 0, 0)),
                pl.BlockSpec((1, _C // 2, _NR), lambda b: (b, 0, 0)),
                pl.BlockSpec((1, _WPI, _K, _NPWP), lambda b: (b, 0, 0, 0)),
            ],
            out_shape=[
                jax.ShapeDtypeStruct((_HB, _C, _N), jnp.float32),
                jax.ShapeDtypeStruct((_HB, _C // 2, _NR), jnp.int32),
                jax.ShapeDtypeStruct((_HB, _WPI, _K, _NPWP), jnp.int32),
            ],
        )(x3, w1, b1, relT, pool)

    mesh = plsc.VectorSubcoreMesh(core_axis_name="c", subcore_axis_name="s")

    def sc(yT, idx):
        return pl.kernel(
            _sc_body, mesh=mesh,
            compiler_params=pltpu.CompilerParams(needs_layout_passes=False),
            out_type=jax.ShapeDtypeStruct((_HB, _WPI, _C, _NPWP),
                                          jnp.float32),
            scratch_types=[
                pltpu.VMEM((_C // 2, _NR), jnp.int32),
                pltpu.VMEM((_K, _NPWP), jnp.int32),
                pltpu.VMEM((_C, _NPWP), jnp.float32),
            ],
        )(yT, idx)

    h_a, yT_a, idx_a = tc1(0)
    h_b, yT_b, idx_b = tc1(_HB)
    acc_a = sc(yT_a, idx_a)
    acc_b = sc(yT_b, idx_b)

    w_specs = [full((2 * _C, 2 * _C)), full((2 * _C, 1)),
               full((2 * _C, 1)), full((2 * _C, 1)),
               full((_C, 2 * _C)), full((_C, 1))]
    w_args = (mr_w, mr_b[:, None], mr_g[:, None], mr_beta[:, None], w2, b2)

    out_a = pl.pallas_call(
        _tc2a_body,
        grid=(_HB,),
        in_specs=[
            pl.BlockSpec((1, _C, _N), lambda b: (b, 0, 0)),
            pl.BlockSpec((1, _WPI, _C, _NPWP), lambda b: (b, 0, 0, 0)),
            pl.BlockSpec((1, _C, _N), lambda b: (b, 0, 0)),
        ] + w_specs,
        out_specs=pl.BlockSpec((1, _C, _N), lambda b: (b, 0, 0)),
        out_shape=jax.ShapeDtypeStruct((_HB, _C, _N), jnp.float32),
    )(h_a, acc_a, x3, *w_args)

    out = pl.pallas_call(
        _tc2b_body,
        grid=(_B,),
        in_specs=[
            pl.BlockSpec((1, _C, _N),
                         lambda b: (jnp.minimum(b, _HB - 1), 0, 0)),
            pl.BlockSpec((1, _C, _N),
                         lambda b: (jnp.maximum(b - _HB, 0), 0, 0)),
            pl.BlockSpec((1, _WPI, _C, _NPWP),
                         lambda b: (jnp.maximum(b - _HB, 0), 0, 0, 0)),
            pl.BlockSpec((1, _C, _N), lambda b: (b, 0, 0)),
        ] + w_specs,
        out_specs=pl.BlockSpec((1, _C, _N), lambda b: (b, 0, 0)),
        out_shape=jax.ShapeDtypeStruct((_B, _C, _N), jnp.float32),
    )(out_a, h_b, acc_b, x3, *w_args)

    return out.reshape(_B, _C, _H, _W)


# speedup vs baseline: 1.4939x; 1.0031x over previous
import math

import jax
import jax.numpy as jnp
import numpy as np
from jax import lax
from jax.experimental import pallas as pl
from jax.experimental.pallas import tpu as pltpu
from jax.experimental.pallas import tpu_sc as plsc

_B, _C, _H, _W = 8, 96, 56, 56
_K = 9
_HR, _WR = 7, 7
_N = _H * _W
_NR = _HR * _WR
_GROUPS = 4
_BIG = 3.0e38

_NC, _NS, _L = 2, 16, 16
_NW = _NC * _NS

_HB = _B // 2
_WPI = _NW // _HB
_NPW = _N // _WPI
_NPWP = -(-_NPW // _L) * _L


def _pos_embed_np(c, h, w):
    d = c // 2
    pe = np.zeros((c, h, w), dtype=np.float32)
    div = np.exp(np.arange(0.0, d, 2) * -(math.log(10000.0) / d))
    pos_w = np.arange(0.0, w)[:, None]
    pos_h = np.arange(0.0, h)[:, None]
    pe[0:d:2, :, :] = np.sin(pos_w * div).T[:, None, :]
    pe[1:d:2, :, :] = np.cos(pos_w * div).T[:, None, :]
    pe[d::2, :, :] = np.sin(pos_h * div).T[:, :, None]
    pe[d + 1::2, :, :] = np.cos(pos_h * div).T[:, :, None]
    return pe


def _constants():
    pos = _pos_embed_np(_C, _H, _W)
    pos_red = pos.reshape(_C, _HR, _H // _HR, _WR, _W // _WR).mean(axis=(2, 4))
    rel = 2.0 * (pos.reshape(_C, -1).T @ pos_red.reshape(_C, -1)) / _C
    relT = np.ascontiguousarray(rel.T).astype(np.float32)
    hh = np.arange(_H)[:, None]
    ww = np.arange(_W)[None, :]
    blk = (hh // (_H // _HR)) * _WR + (ww // (_W // _WR))
    pool = (blk.reshape(_N, 1) == np.arange(_NR)[None, :]).astype(np.float32) / 64.0
    return relT, pool


_RELT_NP, _POOL_NP = _constants()


def _tc1_body(x_ref, w1_ref, b1_ref, relT_ref, pool_ref,
              h_ref, yT_ref, idx_ref):
    x = x_ref[0]
    h = jnp.dot(w1_ref[...], x,
                preferred_element_type=jnp.float32) + b1_ref[...]
    h_ref[0] = h

    y = jnp.dot(h, pool_ref[...], preferred_element_type=jnp.float32)
    yr = y.astype(jnp.bfloat16).reshape(_C // 2, 2, _NR)
    ue = jax.lax.bitcast_convert_type(yr[:, 0, :], jnp.uint16).astype(jnp.uint32)
    uo = jax.lax.bitcast_convert_type(yr[:, 1, :], jnp.uint16).astype(jnp.uint32)
    yT_ref[0] = jax.lax.bitcast_convert_type((uo << 16) | ue, jnp.int32)

    nx = jnp.sqrt(jnp.sum(h * h, axis=0, keepdims=True))
    ny = jnp.sqrt(jnp.sum(y * y, axis=0, keepdims=True))
    ipT = jax.lax.dot_general(y, h, (((0,), (0,)), ((), ())),
                              preferred_element_type=jnp.float32)
    inv_x = 1.0 / (nx + 1e-12)
    inv_y = 1.0 / (ny + 1e-12)
    innerT = ipT * inv_x * inv_y.reshape(_NR, 1)
    sx = (nx * inv_x) ** 2
    sy = (ny * inv_y) ** 2
    distT = 2.0 * innerT - sx - sy.reshape(_NR, 1) + relT_ref[...]

    iota0 = jax.lax.broadcasted_iota(jnp.int32, (_NR, _N), 0)
    d = distT
    rows = []
    for _ in range(_K):
        cur = jnp.max(d, axis=0, keepdims=True)
        first = jnp.min(jnp.where(d >= cur, iota0, _NR), axis=0,
                        keepdims=True)
        rows.append(first)
        d = jnp.where(iota0 == first, -_BIG, d)
    idxmat = jnp.concatenate(rows, axis=0)
    pad = jnp.zeros((_K, _NPWP - _NPW), jnp.int32)
    for q in range(_WPI):
        idx_ref[0, q] = jnp.concatenate(
            [idxmat[:, q * _NPW:(q + 1) * _NPW], pad], axis=1)


def _sc_body(yT_hbm, idx_hbm, out_hbm, yT_v, idx_v, acc_v):
    wid = lax.axis_index("s") * _NC + lax.axis_index("c")
    b = wid // _WPI
    q = wid % _WPI

    pltpu.sync_copy(yT_hbm.at[b], yT_v)
    pltpu.sync_copy(idx_hbm.at[b, q], idx_v)

    cu = 8

    def grp_body(g, carry):
        base = g * _L
        ivs = [idx_v[k, pl.ds(base, _L)] for k in range(_K)]

        def c_body(cc, carry2):
            c0 = cc * cu
            for u in range(cu):
                cp = c0 + u
                csplat = jnp.full((_L,), 0, jnp.int32) + cp
                o = plsc.bitcast(plsc.load_gather(yT_v, [csplat, ivs[0]]),
                                 jnp.bfloat16)
                for k in range(1, _K):
                    o = jnp.maximum(o, plsc.bitcast(
                        plsc.load_gather(yT_v, [csplat, ivs[k]]),
                        jnp.bfloat16))
                ev, od = plsc.unpack(o, format=plsc.PackFormat.INTERLEAVED)
                acc_v[2 * cp, pl.ds(base, _L)] = ev
                acc_v[2 * cp + 1, pl.ds(base, _L)] = od
            return carry2

        return lax.fori_loop(0, (_C // 2) // cu, c_body, carry)

    lax.fori_loop(0, _NPWP // _L, grp_body, 0)
    pltpu.sync_copy(acc_v, out_hbm.at[b, q])


def _tc2_compute(h, acc_ref, x, mrw_ref, mrb_ref, mrg_ref, mrbeta_ref,
                 w2_ref, b2_ref):
    acc = jnp.concatenate(
        [acc_ref[0, q, :, :_NPW] for q in range(_WPI)], axis=1)
    cat = jnp.concatenate([h, acc - h], axis=0)
    g = jnp.dot(mrw_ref[...].astype(jnp.bfloat16), cat.astype(jnp.bfloat16),
                preferred_element_type=jnp.float32) + mrb_ref[...]

    rows = (2 * _C) // _GROUPS
    parts = []
    for gi in range(_GROUPS):
        sub = g[gi * rows:(gi + 1) * rows, :]
        m = jnp.mean(sub)
        dsub = sub - m
        v = jnp.mean(dsub * dsub)
        parts.append(dsub * jax.lax.rsqrt(v + 1e-5))
    gn = jnp.concatenate(parts, axis=0) * mrg_ref[...] + mrbeta_ref[...]
    act = jax.nn.gelu(gn)

    return (jnp.dot(w2_ref[...].astype(jnp.bfloat16),
                    act.astype(jnp.bfloat16),
                    preferred_element_type=jnp.float32)
            + b2_ref[...] + x)


def _tc2a_body(h_ref, acc_ref, x_ref, mrw_ref, mrb_ref, mrg_ref,
               mrbeta_ref, w2_ref, b2_ref, out_ref):
    out_ref[0] = _tc2_compute(h_ref[0], acc_ref, x_ref[0], mrw_ref, mrb_ref,
                              mrg_ref, mrbeta_ref, w2_ref, b2_ref)


def _tc2b_body(outa_ref, h_ref, acc_ref, x_ref, mrw_ref, mrb_ref, mrg_ref,
               mrbeta_ref, w2_ref, b2_ref, out_ref):
    b = pl.program_id(0)

    @pl.when(b < _HB)
    def _copy():
        out_ref[0] = outa_ref[0]

    @pl.when(b >= _HB)
    def _compute():
        out_ref[0] = _tc2_compute(h_ref[0], acc_ref, x_ref[0], mrw_ref,
                                  mrb_ref, mrg_ref, mrbeta_ref, w2_ref,
                                  b2_ref)


def kernel(x, fc1_w, fc1_b, fc1_g, fc1_beta, mr_w, mr_b, mr_g, mr_beta,
           fc2_w, fc2_b, fc2_g, fc2_beta):
    x3 = x.reshape(_B, _C, _N)
    w1 = fc1_g[:, None] * fc1_w
    b1 = (fc1_g * fc1_b + fc1_beta)[:, None]
    w2 = fc2_g[:, None] * fc2_w
    b2 = (fc2_g * fc2_b + fc2_beta)[:, None]
    relT = jnp.asarray(_RELT_NP)
    pool = jnp.asarray(_POOL_NP)

    full = lambda shape: pl.BlockSpec(shape, lambda b: (0,) * len(shape))

    def tc1(off):
        return pl.pallas_call(
            _tc1_body,
            grid=(_HB,),
            in_specs=[
                pl.BlockSpec((1, _C, _N), lambda b: (b + off, 0, 0)),
                full((_C, _C)), full((_C, 1)),
                full((_NR, _N)), full((_N, _NR)),
            ],
            out_specs=[
                pl.BlockSpec((1, _C, _N), lambda b: (b, 0, 0)),
                pl.BlockSpec((1, _C // 2, _NR), lambda b: (b, 0, 0)),
                pl.BlockSpec((1, _WPI, _K, _NPWP), lambda b: (b, 0, 0, 0)),
            ],
            out_shape=[
                jax.ShapeDtypeStruct((_HB, _C, _N), jnp.float32),
                jax.ShapeDtypeStruct((_HB, _C // 2, _NR), jnp.int32),
                jax.ShapeDtypeStruct((_HB, _WPI, _K, _NPWP), jnp.int32),
            ],
        )(x3, w1, b1, relT, pool)

    mesh = plsc.VectorSubcoreMesh(core_axis_name="c", subcore_axis_name="s")

    def sc(yT, idx):
        return pl.kernel(
            _sc_body, mesh=mesh,
            compiler_params=pltpu.CompilerParams(needs_layout_passes=False),
            out_type=jax.ShapeDtypeStruct((_HB, _WPI, _C, _NPWP),
                                          jnp.float32),
            scratch_types=[
                pltpu.VMEM((_C // 2, _NR), jnp.int32),
                pltpu.VMEM((_K, _NPWP), jnp.int32),
                pltpu.VMEM((_C, _NPWP), jnp.float32),
            ],
        )(yT, idx)

    h_a, yT_a, idx_a = tc1(0)
    h_b, yT_b, idx_b = tc1(_HB)
    acc_a = sc(yT_a, idx_a)
    acc_b = sc(yT_b, idx_b)

    w_specs = [full((2 * _C, 2 * _C)), full((2 * _C, 1)),
               full((2 * _C, 1)), full((2 * _C, 1)),
               full((_C, 2 * _C)), full((_C, 1))]
    w_args = (mr_w, mr_b[:, None], mr_g[:, None], mr_beta[:, None], w2, b2)

    out_a = pl.pallas_call(
        _tc2a_body,
        grid=(_HB,),
        in_specs=[
            pl.BlockSpec((1, _C, _N), lambda b: (b, 0, 0)),
            pl.BlockSpec((1, _WPI, _C, _NPWP), lambda b: (b, 0, 0, 0)),
            pl.BlockSpec((1, _C, _N), lambda b: (b, 0, 0)),
        ] + w_specs,
        out_specs=pl.BlockSpec((1, _C, _N), lambda b: (b, 0, 0)),
        out_shape=jax.ShapeDtypeStruct((_HB, _C, _N), jnp.float32),
    )(h_a, acc_a, x3, *w_args)

    out = pl.pallas_call(
        _tc2b_body,
        grid=(_B,),
        in_specs=[
            pl.BlockSpec((1, _C, _N),
                         lambda b: (jnp.minimum(b, _HB - 1), 0, 0)),
            pl.BlockSpec((1, _C, _N),
                         lambda b: (jnp.maximum(b - _HB, 0), 0, 0)),
            pl.BlockSpec((1, _WPI, _C, _NPWP),
                         lambda b: (jnp.maximum(b - _HB, 0), 0, 0, 0)),
            pl.BlockSpec((1, _C, _N), lambda b: (b, 0, 0)),
        ] + w_specs,
        out_specs=pl.BlockSpec((1, _C, _N), lambda b: (b, 0, 0)),
        out_shape=jax.ShapeDtypeStruct((_B, _C, _N), jnp.float32),
    )(out_a, h_b, acc_b, x3, *w_args)

    return out.reshape(_B, _C, _H, _W)
